# Initial kernel scaffold; baseline (speedup 1.0000x reference)
import jax
import jax.numpy as jnp
from jax.experimental import pallas as pl


def kernel(inputs, state, edge_index, bias_1, W_gcn1, b_gcn1, W_gcn2, b_gcn2, linl_w, linl_b, linr_w, linr_b, att, gat_bias):
    raise NotImplementedError("write your pallas kernel here")


# speedup vs baseline: 1.2588x; 1.2588x over previous
import functools

import jax
import jax.numpy as jnp
from jax import lax
from jax.experimental import pallas as pl
from jax.experimental.pallas import tpu as pltpu
from jax.experimental.pallas import tpu_sc as plsc

N = 307
U = 100
UP = 112
IN = 3
B = 64
E = 340
ET = E + N
NG = (ET + 15) // 16
ETP = NG * 16
BN = B * N
ROW_BLOCKS = 8
RB = BN // ROW_BLOCKS
NEG = -1e30


def _tc_pre_body(st_ref, ip_ref, lst_ref, lip_ref, lb_ref, rst_ref, rip_ref,
                 rb_ref, w1ip_ref, b1_ref, w2ip_ref, b2_ref,
                 xl_ref, xr_ref, p1_ref, p2_ref):
    st = st_ref[...]
    ip = ip_ref[...]
    dot = functools.partial(jnp.dot, preferred_element_type=jnp.float32)
    xl_ref[...] = dot(st, lst_ref[...]) + dot(ip, lip_ref[...]) + lb_ref[...]
    xr_ref[...] = dot(st, rst_ref[...]) + dot(ip, rip_ref[...]) + rb_ref[...]
    p1_ref[...] = dot(ip, w1ip_ref[...]) + b1_ref[...]
    p2_ref[...] = dot(ip, w2ip_ref[...]) + b2_ref[...]


def _tc_pre(st2d, ip2d, Lst, Lip, lbp, Rst, Rip, rbp, W1ip, b1p, W2ip, b2p):
    return pl.pallas_call(
        _tc_pre_body,
        grid=(ROW_BLOCKS,),
        in_specs=[
            pl.BlockSpec((RB, U), lambda i: (i, 0)),
            pl.BlockSpec((RB, IN), lambda i: (i, 0)),
            pl.BlockSpec((U, UP), lambda i: (0, 0)),
            pl.BlockSpec((IN, UP), lambda i: (0, 0)),
            pl.BlockSpec((1, UP), lambda i: (0, 0)),
            pl.BlockSpec((U, UP), lambda i: (0, 0)),
            pl.BlockSpec((IN, UP), lambda i: (0, 0)),
            pl.BlockSpec((1, UP), lambda i: (0, 0)),
            pl.BlockSpec((IN, 2 * UP), lambda i: (0, 0)),
            pl.BlockSpec((1, 2 * UP), lambda i: (0, 0)),
            pl.BlockSpec((IN, UP), lambda i: (0, 0)),
            pl.BlockSpec((1, UP), lambda i: (0, 0)),
        ],
        out_specs=[
            pl.BlockSpec((RB, UP), lambda i: (i, 0)),
            pl.BlockSpec((RB, UP), lambda i: (i, 0)),
            pl.BlockSpec((RB, 2 * UP), lambda i: (i, 0)),
            pl.BlockSpec((RB, UP), lambda i: (i, 0)),
        ],
        out_shape=[
            jax.ShapeDtypeStruct((BN, UP), jnp.float32),
            jax.ShapeDtypeStruct((BN, UP), jnp.float32),
            jax.ShapeDtypeStruct((BN, 2 * UP), jnp.float32),
            jax.ShapeDtypeStruct((BN, UP), jnp.float32),
        ],
    )(st2d, ip2d, Lst, Lip, lbp, Rst, Rip, rbp, W1ip, b1p, W2ip, b2p)


def _tc_post_body(x1_ref, p1_ref, p2_ref, w1h_ref, w2h_ref, bv_ref, out_ref):
    dot = functools.partial(jnp.dot, preferred_element_type=jnp.float32)
    st2 = x1_ref[...] + bv_ref[...]
    v = jax.nn.sigmoid(p1_ref[...] + dot(st2, w1h_ref[...]))
    r = v[:, :UP]
    u = v[:, UP:]
    c = jnp.tanh(p2_ref[...] + dot(r * st2, w2h_ref[...]))
    o = u * st2 + (1.0 - u) * c
    out_ref[...] = o[:, :U]


def _tc_post(x1, P1, P2, W1h, W2h, bvec):
    return pl.pallas_call(
        _tc_post_body,
        grid=(ROW_BLOCKS,),
        in_specs=[
            pl.BlockSpec((RB, UP), lambda i: (i, 0)),
            pl.BlockSpec((RB, 2 * UP), lambda i: (i, 0)),
            pl.BlockSpec((RB, UP), lambda i: (i, 0)),
            pl.BlockSpec((UP, 2 * UP), lambda i: (0, 0)),
            pl.BlockSpec((UP, UP), lambda i: (0, 0)),
            pl.BlockSpec((1, UP), lambda i: (0, 0)),
        ],
        out_specs=pl.BlockSpec((RB, U), lambda i: (i, 0)),
        out_shape=jax.ShapeDtypeStruct((BN, U), jnp.float32),
    )(x1, P1, P2, W1h, W2h, bvec)


_info = plsc.get_sparse_core_info()
_NC = _info.num_cores
_NS = _info.num_subcores
_NW = _NC * _NS
_BPW = B // _NW
_DEN = 320


def _sc_edge_body(xl_hbm, xr_hbm, src_hbm, dst_hbm, att_hbm, out_hbm,
                  xl_v, xr_v, out_v, src_v, dst_v, att_v, logit_v, e_v,
                  denom_v):
    cid = lax.axis_index("c")
    sid = lax.axis_index("s")
    wid = sid * _NC + cid
    pltpu.sync_copy(src_hbm, src_v)
    pltpu.sync_copy(dst_hbm, dst_v)
    pltpu.sync_copy(att_hbm, att_v)
    zero16 = jnp.zeros((16,), jnp.float32)

    for bi in range(_BPW):
        b = wid * _BPW + bi
        pltpu.sync_copy(xl_hbm.at[b], xl_v)
        pltpu.sync_copy(xr_hbm.at[b], xr_v)

        def group_a(g, gmax):
            src16 = src_v[pl.ds(g * 16, 16)]
            dst16 = dst_v[pl.ds(g * 16, 16)]
            sbase = src16 * UP
            dbase = dst16 * UP

            def col(c, acc):
                xlc = plsc.load_gather(xl_v, [sbase + c])
                xrc = plsc.load_gather(xr_v, [dbase + c])
                m = xlc + xrc
                m = jnp.where(m > 0, m, 0.2 * m)
                a = att_v[pl.ds(c, 16)][0]
                return acc + m * a

            acc = lax.fori_loop(0, UP, col, zero16)
            lane = g * 16 + lax.iota(jnp.int32, 16)
            lg = jnp.where(lane < ET, acc, NEG)
            logit_v[pl.ds(g * 16, 16)] = lg
            return jnp.maximum(gmax, lg)

        gmaxv = lax.fori_loop(0, NG, group_a, jnp.full((16,), NEG, jnp.float32))
        gmax = jnp.max(gmaxv)

        def zden(i, carry):
            denom_v[pl.ds(i * 16, 16)] = zero16
            return carry

        lax.fori_loop(0, _DEN // 16, zden, 0)

        def group_b(g, carry):
            lg = logit_v[pl.ds(g * 16, 16)]
            e16 = jnp.exp(lg - gmax)
            e_v[pl.ds(g * 16, 16)] = e16
            dst16 = dst_v[pl.ds(g * 16, 16)]
            plsc.addupdate_scatter(denom_v, [dst16], e16)
            return carry

        lax.fori_loop(0, NG, group_b, 0)

        def zout(i, carry):
            out_v[pl.ds(i * 16, 16)] = zero16
            return carry

        lax.fori_loop(0, (N * UP) // 16, zout, 0)

        def group_c(g, carry):
            src16 = src_v[pl.ds(g * 16, 16)]
            dst16 = dst_v[pl.ds(g * 16, 16)]
            e16 = e_v[pl.ds(g * 16, 16)]
            den = plsc.load_gather(denom_v, [dst16])
            alpha = e16 / (den + 1e-16)
            sbase = src16 * UP
            dbase = dst16 * UP

            def col(c, carry2):
                xlc = plsc.load_gather(xl_v, [sbase + c])
                plsc.addupdate_scatter(out_v, [dbase + c], alpha * xlc)
                return carry2

            lax.fori_loop(0, UP, col, 0)
            return carry

        lax.fori_loop(0, NG, group_c, 0)
        pltpu.sync_copy(out_v, out_hbm.at[b])


_sc_edge = functools.partial(
    pl.kernel,
    out_type=jax.ShapeDtypeStruct((B, N * UP), jnp.float32),
    mesh=plsc.VectorSubcoreMesh(core_axis_name="c", subcore_axis_name="s"),
    compiler_params=pltpu.CompilerParams(needs_layout_passes=False),
    scratch_types=[
        pltpu.VMEM((N * UP,), jnp.float32),
        pltpu.VMEM((N * UP,), jnp.float32),
        pltpu.VMEM((N * UP,), jnp.float32),
        pltpu.VMEM((ETP,), jnp.int32),
        pltpu.VMEM((ETP,), jnp.int32),
        pltpu.VMEM((UP + 16,), jnp.float32),
        pltpu.VMEM((ETP,), jnp.float32),
        pltpu.VMEM((ETP,), jnp.float32),
        pltpu.VMEM((_DEN,), jnp.float32),
    ],
)(_sc_edge_body)


def kernel(inputs, state, edge_index, bias_1, W_gcn1, b_gcn1, W_gcn2, b_gcn2,
           linl_w, linl_b, linr_w, linr_b, att, gat_bias):
    ip2d = inputs.reshape(BN, IN)
    st2d = state.reshape(BN, U)
    loops = jnp.arange(N, dtype=edge_index.dtype)
    src = jnp.pad(jnp.concatenate([edge_index[0], loops]), (0, ETP - ET))
    dst = jnp.pad(jnp.concatenate([edge_index[1], loops]), (0, ETP - ET))

    pad1 = lambda v: jnp.pad(v, (0, UP - U))
    row1 = lambda v: v.reshape(1, -1)
    Lst = jnp.pad(linl_w[:U], [(0, 0), (0, UP - U)])
    Lip = jnp.pad(linl_w[U:], [(0, 0), (0, UP - U)])
    Rst = jnp.pad(linr_w[:U], [(0, 0), (0, UP - U)])
    Rip = jnp.pad(linr_w[U:], [(0, 0), (0, UP - U)])
    W1h = jnp.concatenate(
        [jnp.pad(W_gcn1[IN:, :U], [(0, UP - U), (0, UP - U)]),
         jnp.pad(W_gcn1[IN:, U:], [(0, UP - U), (0, UP - U)])], axis=1)
    W1ip = jnp.concatenate(
        [jnp.pad(W_gcn1[:IN, :U], [(0, 0), (0, UP - U)]),
         jnp.pad(W_gcn1[:IN, U:], [(0, 0), (0, UP - U)])], axis=1)
    b1p = jnp.concatenate([pad1(b_gcn1[:U]), pad1(b_gcn1[U:])])
    W2h = jnp.pad(W_gcn2[IN:], [(0, UP - U), (0, UP - U)])
    W2ip = jnp.pad(W_gcn2[:IN], [(0, 0), (0, UP - U)])

    XL, XR, P1, P2 = _tc_pre(st2d, ip2d, Lst, Lip, row1(pad1(linl_b)),
                             Rst, Rip, row1(pad1(linr_b)),
                             W1ip, row1(b1p), W2ip, row1(pad1(b_gcn2)))

    x1 = _sc_edge(XL.reshape(B, N * UP), XR.reshape(B, N * UP),
                  src.astype(jnp.int32), dst.astype(jnp.int32),
                  jnp.pad(att, (0, UP + 16 - U)))

    out = _tc_post(x1.reshape(BN, UP), P1, P2, W1h, W2h,
                   row1(pad1(bias_1 + gat_bias)))
    return out.reshape(B, N * U)


# speedup vs baseline: 1.4373x; 1.1418x over previous
import functools

import jax
import jax.numpy as jnp
from jax import lax
from jax.experimental import pallas as pl
from jax.experimental.pallas import tpu as pltpu
from jax.experimental.pallas import tpu_sc as plsc

N = 307
U = 100
UP = 112
IN = 3
B = 64
E = 340
ET = E + N
NG = (ET + 15) // 16
ETP = NG * 16
BN = B * N
ROW_BLOCKS = 8
RB = BN // ROW_BLOCKS
NEG = -1e30


def _tc_pre_body(st_ref, ip_ref, lst_ref, lip_ref, lb_ref, rst_ref, rip_ref,
                 rb_ref, w1ip_ref, b1_ref, w2ip_ref, b2_ref,
                 xl_ref, xr_ref, p1_ref, p2_ref):
    st = st_ref[...]
    ip = ip_ref[...]
    dot = functools.partial(jnp.dot, preferred_element_type=jnp.float32)
    xl_ref[...] = dot(st, lst_ref[...]) + dot(ip, lip_ref[...]) + lb_ref[...]
    xr_ref[...] = dot(st, rst_ref[...]) + dot(ip, rip_ref[...]) + rb_ref[...]
    p1_ref[...] = dot(ip, w1ip_ref[...]) + b1_ref[...]
    p2_ref[...] = dot(ip, w2ip_ref[...]) + b2_ref[...]


def _tc_pre(st2d, ip2d, Lst, Lip, lbp, Rst, Rip, rbp, W1ip, b1p, W2ip, b2p):
    return pl.pallas_call(
        _tc_pre_body,
        grid=(ROW_BLOCKS,),
        in_specs=[
            pl.BlockSpec((RB, U), lambda i: (i, 0)),
            pl.BlockSpec((RB, IN), lambda i: (i, 0)),
            pl.BlockSpec((U, UP), lambda i: (0, 0)),
            pl.BlockSpec((IN, UP), lambda i: (0, 0)),
            pl.BlockSpec((1, UP), lambda i: (0, 0)),
            pl.BlockSpec((U, UP), lambda i: (0, 0)),
            pl.BlockSpec((IN, UP), lambda i: (0, 0)),
            pl.BlockSpec((1, UP), lambda i: (0, 0)),
            pl.BlockSpec((IN, 2 * UP), lambda i: (0, 0)),
            pl.BlockSpec((1, 2 * UP), lambda i: (0, 0)),
            pl.BlockSpec((IN, UP), lambda i: (0, 0)),
            pl.BlockSpec((1, UP), lambda i: (0, 0)),
        ],
        out_specs=[
            pl.BlockSpec((RB, UP), lambda i: (i, 0)),
            pl.BlockSpec((RB, UP), lambda i: (i, 0)),
            pl.BlockSpec((RB, 2 * UP), lambda i: (i, 0)),
            pl.BlockSpec((RB, UP), lambda i: (i, 0)),
        ],
        out_shape=[
            jax.ShapeDtypeStruct((BN, UP), jnp.float32),
            jax.ShapeDtypeStruct((BN, UP), jnp.float32),
            jax.ShapeDtypeStruct((BN, 2 * UP), jnp.float32),
            jax.ShapeDtypeStruct((BN, UP), jnp.float32),
        ],
    )(st2d, ip2d, Lst, Lip, lbp, Rst, Rip, rbp, W1ip, b1p, W2ip, b2p)


def _tc_post_body(x1_ref, p1_ref, p2_ref, w1h_ref, w2h_ref, bv_ref, out_ref):
    dot = functools.partial(jnp.dot, preferred_element_type=jnp.float32)
    st2 = x1_ref[...] + bv_ref[...]
    v = jax.nn.sigmoid(p1_ref[...] + dot(st2, w1h_ref[...]))
    r = v[:, :UP]
    u = v[:, UP:]
    c = jnp.tanh(p2_ref[...] + dot(r * st2, w2h_ref[...]))
    o = u * st2 + (1.0 - u) * c
    out_ref[...] = o[:, :U]


def _tc_post(x1, P1, P2, W1h, W2h, bvec):
    return pl.pallas_call(
        _tc_post_body,
        grid=(ROW_BLOCKS,),
        in_specs=[
            pl.BlockSpec((RB, UP), lambda i: (i, 0)),
            pl.BlockSpec((RB, 2 * UP), lambda i: (i, 0)),
            pl.BlockSpec((RB, UP), lambda i: (i, 0)),
            pl.BlockSpec((UP, 2 * UP), lambda i: (0, 0)),
            pl.BlockSpec((UP, UP), lambda i: (0, 0)),
            pl.BlockSpec((1, UP), lambda i: (0, 0)),
        ],
        out_specs=pl.BlockSpec((RB, U), lambda i: (i, 0)),
        out_shape=jax.ShapeDtypeStruct((BN, U), jnp.float32),
    )(x1, P1, P2, W1h, W2h, bvec)


_info = plsc.get_sparse_core_info()
_NC = _info.num_cores
_NS = _info.num_subcores
_NW = _NC * _NS
_BPW = B // _NW
_DEN = 320


def _sc_edge_body(xl_hbm, xr_hbm, src_hbm, dst_hbm, att_hbm, out_hbm,
                  xl_v, xr_v, out_v, src_v, dst_v, att_v, logit_v, e_v,
                  denom_v):
    cid = lax.axis_index("c")
    sid = lax.axis_index("s")
    wid = sid * _NC + cid
    pltpu.sync_copy(src_hbm, src_v)
    pltpu.sync_copy(dst_hbm, dst_v)
    pltpu.sync_copy(att_hbm, att_v)
    zero16 = jnp.zeros((16,), jnp.float32)

    for bi in range(_BPW):
        b = wid * _BPW + bi
        pltpu.sync_copy(xl_hbm.at[b], xl_v)
        pltpu.sync_copy(xr_hbm.at[b], xr_v)

        def group_a(g, gmax):
            src16 = src_v[pl.ds(g * 16, 16)]
            dst16 = dst_v[pl.ds(g * 16, 16)]
            sbase = src16 * UP
            dbase = dst16 * UP

            def chunk_a(cu, accs):
                attc = att_v[pl.ds(cu * 16, 16)]
                bs = sbase + cu * 16
                bd = dbase + cu * 16
                outs = list(accs)
                for j in range(16):
                    xlc = plsc.load_gather(xl_v, [bs + j])
                    xrc = plsc.load_gather(xr_v, [bd + j])
                    m = xlc + xrc
                    m = jnp.maximum(m, 0.2 * m)
                    outs[j % 4] = outs[j % 4] + m * attc[j]
                return tuple(outs)

            a0, a1, a2, a3 = lax.fori_loop(0, UP // 16, chunk_a, (zero16,) * 4)
            acc = (a0 + a1) + (a2 + a3)
            lane = g * 16 + lax.iota(jnp.int32, 16)
            lg = jnp.where(lane < ET, acc, NEG)
            logit_v[pl.ds(g * 16, 16)] = lg
            return jnp.maximum(gmax, lg)

        gmaxv = lax.fori_loop(0, NG, group_a, jnp.full((16,), NEG, jnp.float32))
        gmax = jnp.max(gmaxv)

        for i in range(_DEN // 16):
            denom_v[pl.ds(i * 16, 16)] = zero16

        def group_b(g, carry):
            lg = logit_v[pl.ds(g * 16, 16)]
            e16 = jnp.exp(lg - gmax)
            e_v[pl.ds(g * 16, 16)] = e16
            dst16 = dst_v[pl.ds(g * 16, 16)]
            plsc.addupdate_scatter(denom_v, [dst16], e16)
            return carry

        lax.fori_loop(0, NG, group_b, 0)

        def zout(i, carry):
            for j in range(UP // 16):
                out_v[pl.ds(i * UP + j * 16, 16)] = zero16
            return carry

        lax.fori_loop(0, N, zout, 0)

        def group_c(g, carry):
            src16 = src_v[pl.ds(g * 16, 16)]
            dst16 = dst_v[pl.ds(g * 16, 16)]
            e16 = e_v[pl.ds(g * 16, 16)]
            den = plsc.load_gather(denom_v, [dst16])
            alpha = e16 / (den + 1e-16)
            sbase = src16 * UP
            dbase = dst16 * UP

            def chunk_c(cu, carry2):
                bs = sbase + cu * 16
                bd = dbase + cu * 16
                for j in range(16):
                    xlc = plsc.load_gather(xl_v, [bs + j])
                    plsc.addupdate_scatter(out_v, [bd + j], alpha * xlc)
                return carry2

            lax.fori_loop(0, UP // 16, chunk_c, 0)
            return carry

        lax.fori_loop(0, NG, group_c, 0)
        pltpu.sync_copy(out_v, out_hbm.at[b])


_sc_edge = functools.partial(
    pl.kernel,
    out_type=jax.ShapeDtypeStruct((B, N * UP), jnp.float32),
    mesh=plsc.VectorSubcoreMesh(core_axis_name="c", subcore_axis_name="s"),
    compiler_params=pltpu.CompilerParams(needs_layout_passes=False),
    scratch_types=[
        pltpu.VMEM((N * UP,), jnp.float32),
        pltpu.VMEM((N * UP,), jnp.float32),
        pltpu.VMEM((N * UP,), jnp.float32),
        pltpu.VMEM((ETP,), jnp.int32),
        pltpu.VMEM((ETP,), jnp.int32),
        pltpu.VMEM((UP + 16,), jnp.float32),
        pltpu.VMEM((ETP,), jnp.float32),
        pltpu.VMEM((ETP,), jnp.float32),
        pltpu.VMEM((_DEN,), jnp.float32),
    ],
)(_sc_edge_body)


def kernel(inputs, state, edge_index, bias_1, W_gcn1, b_gcn1, W_gcn2, b_gcn2,
           linl_w, linl_b, linr_w, linr_b, att, gat_bias):
    ip2d = inputs.reshape(BN, IN)
    st2d = state.reshape(BN, U)
    loops = jnp.arange(N, dtype=edge_index.dtype)
    src = jnp.pad(jnp.concatenate([edge_index[0], loops]), (0, ETP - ET))
    dst = jnp.pad(jnp.concatenate([edge_index[1], loops]), (0, ETP - ET))

    pad1 = lambda v: jnp.pad(v, (0, UP - U))
    row1 = lambda v: v.reshape(1, -1)
    Lst = jnp.pad(linl_w[:U], [(0, 0), (0, UP - U)])
    Lip = jnp.pad(linl_w[U:], [(0, 0), (0, UP - U)])
    Rst = jnp.pad(linr_w[:U], [(0, 0), (0, UP - U)])
    Rip = jnp.pad(linr_w[U:], [(0, 0), (0, UP - U)])
    W1h = jnp.concatenate(
        [jnp.pad(W_gcn1[IN:, :U], [(0, UP - U), (0, UP - U)]),
         jnp.pad(W_gcn1[IN:, U:], [(0, UP - U), (0, UP - U)])], axis=1)
    W1ip = jnp.concatenate(
        [jnp.pad(W_gcn1[:IN, :U], [(0, 0), (0, UP - U)]),
         jnp.pad(W_gcn1[:IN, U:], [(0, 0), (0, UP - U)])], axis=1)
    b1p = jnp.concatenate([pad1(b_gcn1[:U]), pad1(b_gcn1[U:])])
    W2h = jnp.pad(W_gcn2[IN:], [(0, UP - U), (0, UP - U)])
    W2ip = jnp.pad(W_gcn2[:IN], [(0, 0), (0, UP - U)])

    XL, XR, P1, P2 = _tc_pre(st2d, ip2d, Lst, Lip, row1(pad1(linl_b)),
                             Rst, Rip, row1(pad1(linr_b)),
                             W1ip, row1(b1p), W2ip, row1(pad1(b_gcn2)))

    x1 = _sc_edge(XL.reshape(B, N * UP), XR.reshape(B, N * UP),
                  src.astype(jnp.int32), dst.astype(jnp.int32),
                  jnp.pad(att, (0, UP + 16 - U)))

    out = _tc_post(x1.reshape(BN, UP), P1, P2, W1h, W2h,
                   row1(pad1(bias_1 + gat_bias)))
    return out.reshape(B, N * U)


# speedup vs baseline: 1.7370x; 1.2085x over previous
import functools

import jax
import jax.numpy as jnp
from jax import lax
from jax.experimental import pallas as pl
from jax.experimental.pallas import tpu as pltpu
from jax.experimental.pallas import tpu_sc as plsc

N = 307
U = 100
UP = 112
IN = 3
B = 64
E = 340
ET = E + N
NG = (ET + 15) // 16
ETP = NG * 16
BN = B * N
ROW_BLOCKS = 8
RB = BN // ROW_BLOCKS
NEG = -1e30
UPS = 113
NR = 312
SCW = NR * UPS
ZCH = 543


def _tc_pre_body(st_ref, ip_ref, lst_ref, lip_ref, lb_ref, rst_ref, rip_ref,
                 rb_ref, w1ip_ref, b1_ref, w2ip_ref, b2_ref,
                 xl_ref, xr_ref, p1_ref, p2_ref):
    st = st_ref[...]
    ip = ip_ref[...]
    dot = functools.partial(jnp.dot, preferred_element_type=jnp.float32)
    xl_ref[...] = dot(st, lst_ref[...]) + dot(ip, lip_ref[...]) + lb_ref[...]
    xr_ref[...] = dot(st, rst_ref[...]) + dot(ip, rip_ref[...]) + rb_ref[...]
    p1_ref[...] = dot(ip, w1ip_ref[...]) + b1_ref[...]
    p2_ref[...] = dot(ip, w2ip_ref[...]) + b2_ref[...]


def _tc_pre(st2d, ip2d, Lst, Lip, lbp, Rst, Rip, rbp, W1ip, b1p, W2ip, b2p):
    return pl.pallas_call(
        _tc_pre_body,
        grid=(ROW_BLOCKS,),
        in_specs=[
            pl.BlockSpec((RB, U), lambda i: (i, 0)),
            pl.BlockSpec((RB, IN), lambda i: (i, 0)),
            pl.BlockSpec((U, UP), lambda i: (0, 0)),
            pl.BlockSpec((IN, UP), lambda i: (0, 0)),
            pl.BlockSpec((1, UP), lambda i: (0, 0)),
            pl.BlockSpec((U, UP), lambda i: (0, 0)),
            pl.BlockSpec((IN, UP), lambda i: (0, 0)),
            pl.BlockSpec((1, UP), lambda i: (0, 0)),
            pl.BlockSpec((IN, 2 * UP), lambda i: (0, 0)),
            pl.BlockSpec((1, 2 * UP), lambda i: (0, 0)),
            pl.BlockSpec((IN, UP), lambda i: (0, 0)),
            pl.BlockSpec((1, UP), lambda i: (0, 0)),
        ],
        out_specs=[
            pl.BlockSpec((RB, UP), lambda i: (i, 0)),
            pl.BlockSpec((RB, UP), lambda i: (i, 0)),
            pl.BlockSpec((RB, 2 * UP), lambda i: (i, 0)),
            pl.BlockSpec((RB, UP), lambda i: (i, 0)),
        ],
        out_shape=[
            jax.ShapeDtypeStruct((BN, UP), jnp.float32),
            jax.ShapeDtypeStruct((BN, UP), jnp.float32),
            jax.ShapeDtypeStruct((BN, 2 * UP), jnp.float32),
            jax.ShapeDtypeStruct((BN, UP), jnp.float32),
        ],
    )(st2d, ip2d, Lst, Lip, lbp, Rst, Rip, rbp, W1ip, b1p, W2ip, b2p)


def _tc_post_body(x1_ref, p1_ref, p2_ref, w1h_ref, w2h_ref, bv_ref, out_ref):
    dot = functools.partial(jnp.dot, preferred_element_type=jnp.float32)
    st2 = x1_ref[...] + bv_ref[...]
    v = jax.nn.sigmoid(p1_ref[...] + dot(st2, w1h_ref[...]))
    r = v[:, :UP]
    u = v[:, UP:]
    c = jnp.tanh(p2_ref[...] + dot(r * st2, w2h_ref[...]))
    o = u * st2 + (1.0 - u) * c
    out_ref[...] = o[:, :U]


def _tc_post(x1, P1, P2, W1h, W2h, bvec):
    return pl.pallas_call(
        _tc_post_body,
        grid=(ROW_BLOCKS,),
        in_specs=[
            pl.BlockSpec((RB, UP), lambda i: (i, 0)),
            pl.BlockSpec((RB, 2 * UP), lambda i: (i, 0)),
            pl.BlockSpec((RB, UP), lambda i: (i, 0)),
            pl.BlockSpec((UP, 2 * UP), lambda i: (0, 0)),
            pl.BlockSpec((UP, UP), lambda i: (0, 0)),
            pl.BlockSpec((1, UP), lambda i: (0, 0)),
        ],
        out_specs=pl.BlockSpec((RB, U), lambda i: (i, 0)),
        out_shape=jax.ShapeDtypeStruct((BN, U), jnp.float32),
    )(x1, P1, P2, W1h, W2h, bvec)


_info = plsc.get_sparse_core_info()
_NC = _info.num_cores
_NS = _info.num_subcores
_NW = _NC * _NS
_BPW = B // _NW
_DEN = 320


def _sc_edge_body(xl_hbm, xr_hbm, src_hbm, dst_hbm, att_hbm, out_hbm,
                  xl_v, xr_v, out_v, src_v, dst_v, att_v, logit_v, e_v,
                  denom_v):
    cid = lax.axis_index("c")
    sid = lax.axis_index("s")
    wid = sid * _NC + cid
    pltpu.sync_copy(src_hbm, src_v)
    pltpu.sync_copy(dst_hbm, dst_v)
    pltpu.sync_copy(att_hbm, att_v)
    zero16 = jnp.zeros((16,), jnp.float32)

    for bi in range(_BPW):
        b = wid * _BPW + bi
        pltpu.sync_copy(xl_hbm.at[b], xl_v)
        pltpu.sync_copy(xr_hbm.at[b], xr_v)

        def group_a(g, gmax):
            src16 = src_v[pl.ds(g * 16, 16)]
            dst16 = dst_v[pl.ds(g * 16, 16)]
            sbase = src16 * UPS
            dbase = dst16 * UPS

            def chunk_a(cu, accs):
                attc = att_v[pl.ds(cu * 16, 16)]
                bs = sbase + cu * 16
                bd = dbase + cu * 16
                outs = list(accs)
                for j in range(16):
                    xlc = plsc.load_gather(xl_v, [bs + j])
                    xrc = plsc.load_gather(xr_v, [bd + j])
                    m = xlc + xrc
                    m = jnp.maximum(m, 0.2 * m)
                    outs[j % 4] = outs[j % 4] + m * attc[j]
                return tuple(outs)

            a0, a1, a2, a3 = lax.fori_loop(0, UP // 16, chunk_a, (zero16,) * 4)
            acc = (a0 + a1) + (a2 + a3)
            lane = g * 16 + lax.iota(jnp.int32, 16)
            lg = jnp.where(lane < ET, acc, NEG)
            logit_v[pl.ds(g * 16, 16)] = lg
            return jnp.maximum(gmax, lg)

        gmaxv = lax.fori_loop(0, NG, group_a, jnp.full((16,), NEG, jnp.float32))
        gmax = jnp.max(gmaxv)

        for i in range(_DEN // 16):
            denom_v[pl.ds(i * 16, 16)] = zero16

        def group_b(g, carry):
            lg = logit_v[pl.ds(g * 16, 16)]
            e16 = jnp.exp(lg - gmax)
            e_v[pl.ds(g * 16, 16)] = e16
            dst16 = dst_v[pl.ds(g * 16, 16)]
            plsc.addupdate_scatter(denom_v, [dst16], e16)
            return carry

        lax.fori_loop(0, NG, group_b, 0)

        def zout(i, carry):
            for j in range(4):
                out_v[pl.ds((i * 4 + j) * 16, 16)] = zero16
            return carry

        lax.fori_loop(0, ZCH, zout, 0)

        def group_c(g, carry):
            src16 = src_v[pl.ds(g * 16, 16)]
            dst16 = dst_v[pl.ds(g * 16, 16)]
            e16 = e_v[pl.ds(g * 16, 16)]
            den = plsc.load_gather(denom_v, [dst16])
            alpha = e16 / (den + 1e-16)
            sbase = src16 * UPS
            dbase = dst16 * UPS

            def chunk_c(cu, carry2):
                bs = sbase + cu * 16
                bd = dbase + cu * 16
                for j in range(16):
                    xlc = plsc.load_gather(xl_v, [bs + j])
                    plsc.addupdate_scatter(out_v, [bd + j], alpha * xlc)
                return carry2

            lax.fori_loop(0, UP // 16, chunk_c, 0)
            return carry

        lax.fori_loop(0, NG, group_c, 0)
        pltpu.sync_copy(out_v, out_hbm.at[b])


_sc_edge = functools.partial(
    pl.kernel,
    out_type=jax.ShapeDtypeStruct((B, SCW), jnp.float32),
    mesh=plsc.VectorSubcoreMesh(core_axis_name="c", subcore_axis_name="s"),
    compiler_params=pltpu.CompilerParams(needs_layout_passes=False),
    scratch_types=[
        pltpu.VMEM((SCW,), jnp.float32),
        pltpu.VMEM((SCW,), jnp.float32),
        pltpu.VMEM((SCW,), jnp.float32),
        pltpu.VMEM((ETP,), jnp.int32),
        pltpu.VMEM((ETP,), jnp.int32),
        pltpu.VMEM((UP + 16,), jnp.float32),
        pltpu.VMEM((ETP,), jnp.float32),
        pltpu.VMEM((ETP,), jnp.float32),
        pltpu.VMEM((_DEN,), jnp.float32),
    ],
)(_sc_edge_body)


def kernel(inputs, state, edge_index, bias_1, W_gcn1, b_gcn1, W_gcn2, b_gcn2,
           linl_w, linl_b, linr_w, linr_b, att, gat_bias):
    ip2d = inputs.reshape(BN, IN)
    st2d = state.reshape(BN, U)
    loops = jnp.arange(N, dtype=edge_index.dtype)
    src = jnp.pad(jnp.concatenate([edge_index[0], loops]), (0, ETP - ET))
    dst = jnp.pad(jnp.concatenate([edge_index[1], loops]), (0, ETP - ET))

    pad1 = lambda v: jnp.pad(v, (0, UP - U))
    row1 = lambda v: v.reshape(1, -1)
    Lst = jnp.pad(linl_w[:U], [(0, 0), (0, UP - U)])
    Lip = jnp.pad(linl_w[U:], [(0, 0), (0, UP - U)])
    Rst = jnp.pad(linr_w[:U], [(0, 0), (0, UP - U)])
    Rip = jnp.pad(linr_w[U:], [(0, 0), (0, UP - U)])
    W1h = jnp.concatenate(
        [jnp.pad(W_gcn1[IN:, :U], [(0, UP - U), (0, UP - U)]),
         jnp.pad(W_gcn1[IN:, U:], [(0, UP - U), (0, UP - U)])], axis=1)
    W1ip = jnp.concatenate(
        [jnp.pad(W_gcn1[:IN, :U], [(0, 0), (0, UP - U)]),
         jnp.pad(W_gcn1[:IN, U:], [(0, 0), (0, UP - U)])], axis=1)
    b1p = jnp.concatenate([pad1(b_gcn1[:U]), pad1(b_gcn1[U:])])
    W2h = jnp.pad(W_gcn2[IN:], [(0, UP - U), (0, UP - U)])
    W2ip = jnp.pad(W_gcn2[:IN], [(0, 0), (0, UP - U)])

    XL, XR, P1, P2 = _tc_pre(st2d, ip2d, Lst, Lip, row1(pad1(linl_b)),
                             Rst, Rip, row1(pad1(linr_b)),
                             W1ip, row1(b1p), W2ip, row1(pad1(b_gcn2)))

    scpad = lambda a: jnp.pad(a.reshape(B, N, UP),
                              [(0, 0), (0, NR - N), (0, UPS - UP)]
                              ).reshape(B, SCW)
    x1 = _sc_edge(scpad(XL), scpad(XR),
                  src.astype(jnp.int32), dst.astype(jnp.int32),
                  jnp.pad(att, (0, UP + 16 - U)))

    x1u = x1.reshape(B, NR, UPS)[:, :N, :UP].reshape(BN, UP)
    out = _tc_post(x1u, P1, P2, W1h, W2h,
                   row1(pad1(bias_1 + gat_bias)))
    return out.reshape(B, N * U)
